# baseline (speedup 1.0000x reference)
import functools

import jax
import jax.numpy as jnp
from jax import lax
from jax.experimental import pallas as pl
from jax.experimental.pallas import tpu as pltpu
from jax.experimental.pallas import tpu_sc as plsc

B = 4096
L = 50
LH = L // 2
D = 64
GC = 128


def _sc_gather(table, hu_flat, nodes, utable):
    info = plsc.get_sparse_core_info()
    nw = info.num_cores * info.num_subcores
    n = hu_flat.shape[0]
    npw = n // nw
    nch = npw // GC
    bsz = n // L
    upw = bsz // nw

    @functools.partial(
        pl.kernel,
        mesh=plsc.VectorSubcoreMesh(core_axis_name="c", subcore_axis_name="s"),
        compiler_params=pltpu.CompilerParams(use_tc_tiling_on_sc=False),
        out_type=[
            jax.ShapeDtypeStruct((n, D), jnp.float32),
            jax.ShapeDtypeStruct((bsz, D), jnp.float32),
        ],
        scratch_types=[
            pltpu.VMEM((npw,), jnp.int32),
            pltpu.VMEM((GC, D), jnp.float32),
            pltpu.VMEM((GC, D), jnp.float32),
            pltpu.VMEM((upw,), jnp.int32),
            pltpu.VMEM((upw, D), jnp.float32),
            pltpu.SemaphoreType.DMA,
            pltpu.SemaphoreType.DMA,
        ],
    )
    def k(table_hbm, hu_hbm, nodes_hbm, utable_hbm, out_hbm, uout_hbm,
          idx_v, rb0, rb1, nidx_v, urows_v, sem0, sem1):
        wid = lax.axis_index("s") * info.num_cores + lax.axis_index("c")
        r0 = wid * npw

        pltpu.sync_copy(hu_hbm.at[pl.ds(r0, npw)], idx_v)

        pltpu.async_copy(table_hbm.at[idx_v.at[pl.ds(0, GC)]], rb0, sem0)

        def gbody(j, carry):
            k0 = 2 * j
            pltpu.async_copy(
                table_hbm.at[idx_v.at[pl.ds((k0 + 1) * GC, GC)]], rb1, sem1)
            pltpu.make_async_copy(
                table_hbm.at[idx_v.at[pl.ds(k0 * GC, GC)]], rb0, sem0).wait()
            pltpu.sync_copy(rb0, out_hbm.at[pl.ds(r0 + k0 * GC, GC)])

            @pl.when(j < (nch // 2 - 1))
            def _():
                pltpu.async_copy(
                    table_hbm.at[idx_v.at[pl.ds((k0 + 2) * GC, GC)]], rb0,
                    sem0)

            pltpu.make_async_copy(
                table_hbm.at[idx_v.at[pl.ds((k0 + 1) * GC, GC)]], rb1,
                sem1).wait()
            pltpu.sync_copy(rb1, out_hbm.at[pl.ds(r0 + (k0 + 1) * GC, GC)])
            return carry

        lax.fori_loop(0, nch // 2, gbody, 0)

        pltpu.sync_copy(nodes_hbm.at[pl.ds(wid * upw, upw)], nidx_v)
        pltpu.async_copy(utable_hbm.at[nidx_v], urows_v, sem0).wait()
        pltpu.sync_copy(urows_v, uout_hbm.at[pl.ds(wid * upw, upw)])

    return k(table, hu_flat, nodes, utable)


def _bd(x):
    z = jnp.zeros((D, D), jnp.float32)
    return jnp.concatenate(
        [jnp.concatenate([x, z], axis=1), jnp.concatenate([z, x], axis=1)],
        axis=0)


def _tc_body(g_ref, hr_ref, u_ref, r2e_ref, W1_ref, b1_ref, W2_ref, b2_ref,
             A1_ref, a1b_ref, A2_ref, a2b_ref, A3_ref, out_ref):
    bb = out_ref.shape[0]
    r = LH * bb
    dn = (((1,), (1,)), ((), ()))
    f32 = jnp.float32

    lane = lax.broadcasted_iota(jnp.int32, (2, 2 * D), 1)
    half = lax.broadcasted_iota(jnp.int32, (2, 2 * D), 0)
    Sm = jnp.where((lane < D) == (half == 0), 1.0, 0.0).astype(f32)

    r_tab = lax.dot_general(r2e_ref[...], W1_ref[:, D:], dn,
                            preferred_element_type=f32) + b1_ref[...]
    z5 = jnp.zeros((5, D), f32)
    Rt = jnp.concatenate(
        [jnp.concatenate([r_tab, z5], axis=1),
         jnp.concatenate([z5, r_tab], axis=1)], axis=0)
    col = lax.broadcasted_iota(jnp.int32, (2, 10), 1)
    hh = lax.broadcasted_iota(jnp.int32, (2, 10), 0)
    E = jnp.where((col < 5) == (hh == 0), 1.0, 0.0).astype(f32)
    rvals = jnp.where(col[:1] < 5, col[:1], col[:1] - 5).astype(f32)
    hf = hr_ref[...].astype(f32).reshape(r, 2)
    hrep = lax.dot_general(hf, E, (((1,), (0,)), ((), ())),
                           preferred_element_type=f32)
    oh = jnp.where(hrep == rvals, 1.0, 0.0).astype(f32)
    rc = lax.dot_general(oh, Rt, (((1,), (0,)), ((), ())),
                         preferred_element_type=f32)

    gf = g_ref[...].reshape(r, 2 * D)
    x1 = jnp.maximum(
        lax.dot_general(gf, _bd(W1_ref[:, :D]), dn, preferred_element_type=f32)
        + rc, 0.0)
    b2d = jnp.concatenate([b2_ref[...], b2_ref[...]], axis=1)
    oh_ = jnp.maximum(
        lax.dot_general(x1, _bd(W2_ref[...]), dn, preferred_element_type=f32)
        + b2d, 0.0)

    p = lax.dot_general(oh_, _bd(A1_ref[:, :D]), dn, preferred_element_type=f32)
    uc_h = lax.dot_general(u_ref[...], A1_ref[:, D:], dn,
                           preferred_element_type=f32) + a1b_ref[...]
    uc = jnp.concatenate([uc_h, uc_h], axis=1)
    a1 = jnp.maximum(p.reshape(bb, LH, 2 * D) + uc[:, None, :], 0.0)
    a2bd = jnp.concatenate([a2b_ref[...], a2b_ref[...]], axis=1)
    a2 = jnp.maximum(
        lax.dot_general(a1.reshape(r, 2 * D), _bd(A2_ref[...]), dn,
                        preferred_element_type=f32) + a2bd, 0.0)

    A3d = jnp.concatenate([A3_ref[...], A3_ref[...]], axis=1)
    t2 = lax.dot_general(a2 * A3d, Sm, dn, preferred_element_type=f32)
    t3 = t2.reshape(bb, LH, 2)
    m = jnp.max(jnp.max(t3, axis=2, keepdims=True), axis=1, keepdims=True)
    e = jnp.exp(t3 - m)
    s = jnp.sum(jnp.sum(e, axis=2, keepdims=True), axis=1, keepdims=True)
    w3 = e / s
    wf = lax.dot_general(w3.reshape(r, 2), Sm, (((1,), (0,)), ((), ())),
                         preferred_element_type=f32)
    z = jnp.sum((oh_ * wf).reshape(bb, LH, 2 * D), axis=1)
    out_ref[...] = z[:, :D] + z[:, D:]


def _tc_compute(g3, hrp, urep, r2e_w, W1, b1, W2, b2, A1, a1b, A2, a2b, A3):
    bb = 128
    bsz = g3.shape[0]
    grid = bsz // bb
    full = lambda shape: pl.BlockSpec(shape, lambda i: tuple(0 for _ in shape))
    return pl.pallas_call(
        _tc_body,
        grid=(grid,),
        in_specs=[
            pl.BlockSpec((bb, LH, 2 * D), lambda i: (i, 0, 0)),
            pl.BlockSpec((bb, LH, 2), lambda i: (i, 0, 0)),
            pl.BlockSpec((bb, D), lambda i: (i, 0)),
            full((5, D)),
            full((D, 2 * D)),
            full((1, D)),
            full((D, D)),
            full((1, D)),
            full((D, 2 * D)),
            full((1, D)),
            full((D, D)),
            full((1, D)),
            full((1, D)),
        ],
        out_specs=pl.BlockSpec((bb, D), lambda i: (i, 0)),
        out_shape=jax.ShapeDtypeStruct((bsz, D), jnp.float32),
    )(g3, hrp, urep, r2e_w, W1, b1, W2, b2, A1, a1b, A2, a2b, A3)


def kernel(nodes, history_uv, history_r, history_uvt, v2e_w, u2e_w, r2e_w,
           t2e_w, W1, b1, W2, b2, A1, a1b, A2, a2b, A3, a3b):
    del history_uvt, t2e_w, a3b
    nc = 2
    bsz = B // nc
    hu_i = history_uv.astype(jnp.int32)
    hr_i = history_r.astype(jnp.int32)
    nodes_i = nodes.astype(jnp.int32)
    outs = []
    for c in range(nc):
        sl = slice(c * bsz, (c + 1) * bsz)
        e_flat, urep = _sc_gather(
            v2e_w, hu_i[sl].reshape(-1), nodes_i[sl], u2e_w)
        g3 = e_flat.reshape(bsz, LH, 2 * D)
        hrp = hr_i[sl].reshape(bsz, LH, 2)
        outs.append(_tc_compute(
            g3, hrp, urep, r2e_w, W1,
            b1.reshape(1, D), W2, b2.reshape(1, D),
            A1, a1b.reshape(1, D), A2, a2b.reshape(1, D), A3))
    return jnp.concatenate(outs, axis=0)


# speedup vs baseline: 2.4004x; 2.4004x over previous
import functools

import jax
import jax.numpy as jnp
from jax import lax
from jax.experimental import pallas as pl
from jax.experimental.pallas import tpu as pltpu
from jax.experimental.pallas import tpu_sc as plsc

B = 4096
L = 50
D = 64


def _sc_gather(table, hu_pages, nodes, utable):
    info = plsc.get_sparse_core_info()
    nw = info.num_cores * info.num_subcores
    ch = hu_pages.shape[2]
    bsz = nw * ch
    npw = bsz // nw

    @functools.partial(
        pl.kernel,
        mesh=plsc.VectorSubcoreMesh(core_axis_name="c", subcore_axis_name="s"),
        compiler_params=pltpu.CompilerParams(use_tc_tiling_on_sc=False),
        out_type=[
            jax.ShapeDtypeStruct((L * bsz, D), jnp.float32),
            jax.ShapeDtypeStruct((bsz, D), jnp.float32),
        ],
        scratch_types=[
            pltpu.VMEM((L, ch), jnp.int32),
            pltpu.VMEM((ch, D), jnp.float32),
            pltpu.VMEM((ch, D), jnp.float32),
            pltpu.VMEM((npw,), jnp.int32),
            pltpu.VMEM((npw, D), jnp.float32),
            pltpu.SemaphoreType.DMA,
            pltpu.SemaphoreType.DMA,
        ],
    )
    def k(table_hbm, hu_hbm, nodes_hbm, utable_hbm, out_hbm, uout_hbm,
          idxt_v, rb0, rb1, nidx_v, urows_v, sem0, sem1):
        wid = lax.axis_index("s") * info.num_cores + lax.axis_index("c")
        b0 = wid * ch

        pltpu.sync_copy(hu_hbm.at[wid], idxt_v)

        pltpu.async_copy(table_hbm.at[idxt_v.at[0]], rb0, sem0)

        def gbody(j, carry):
            l0 = 2 * j
            pltpu.async_copy(table_hbm.at[idxt_v.at[l0 + 1]], rb1, sem1)
            pltpu.make_async_copy(table_hbm.at[idxt_v.at[l0]], rb0, sem0).wait()
            pltpu.sync_copy(rb0, out_hbm.at[pl.ds(l0 * bsz + b0, ch)])

            @pl.when(j < (L // 2 - 1))
            def _():
                pltpu.async_copy(table_hbm.at[idxt_v.at[l0 + 2]], rb0, sem0)

            pltpu.make_async_copy(table_hbm.at[idxt_v.at[l0 + 1]], rb1,
                                  sem1).wait()
            pltpu.sync_copy(rb1, out_hbm.at[pl.ds((l0 + 1) * bsz + b0, ch)])
            return carry

        lax.fori_loop(0, L // 2, gbody, 0)

        pltpu.sync_copy(nodes_hbm.at[pl.ds(wid * npw, npw)], nidx_v)
        pltpu.async_copy(utable_hbm.at[nidx_v], urows_v, sem0).wait()
        pltpu.sync_copy(urows_v, uout_hbm.at[pl.ds(wid * npw, npw)])

    return k(table, hu_pages, nodes, utable)


def _bd(x):
    z = jnp.zeros((D, D), jnp.float32)
    return jnp.concatenate(
        [jnp.concatenate([x, z], axis=1), jnp.concatenate([z, x], axis=1)],
        axis=0)


def _tc_body(g_ref, hrp_ref, u_ref, r2e_ref, W1_ref, b1_ref, W2_ref, b2_ref,
             A1_ref, a1b_ref, A2_ref, a2b_ref, A3_ref, out_ref):
    bp = out_ref.shape[0]
    r = L * bp
    dn = (((1,), (1,)), ((), ()))
    f32 = jnp.float32

    lane = lax.broadcasted_iota(jnp.int32, (2, 2 * D), 1)
    half = lax.broadcasted_iota(jnp.int32, (2, 2 * D), 0)
    Sm = jnp.where((lane < D) == (half == 0), 1.0, 0.0).astype(f32)

    r_tab = lax.dot_general(r2e_ref[...], W1_ref[:, D:], dn,
                            preferred_element_type=f32) + b1_ref[...]
    z5 = jnp.zeros((5, D), f32)
    Rt = jnp.concatenate(
        [jnp.concatenate([r_tab, z5], axis=1),
         jnp.concatenate([z5, r_tab], axis=1)], axis=0)
    col = lax.broadcasted_iota(jnp.int32, (2, 10), 1)
    hh = lax.broadcasted_iota(jnp.int32, (2, 10), 0)
    E = jnp.where((col < 5) == (hh == 0), 1.0, 0.0).astype(f32)
    rvals = jnp.where(col[:1] < 5, col[:1], col[:1] - 5).astype(f32)
    hf = jnp.transpose(hrp_ref[...].astype(f32), (2, 0, 1)).reshape(r, 2)
    hrep = lax.dot_general(hf, E, (((1,), (0,)), ((), ())),
                           preferred_element_type=f32)
    oh = jnp.where(hrep == rvals, 1.0, 0.0).astype(f32)
    rc = lax.dot_general(oh, Rt, (((1,), (0,)), ((), ())),
                         preferred_element_type=f32)

    gf = g_ref[...].reshape(r, 2 * D)
    x1 = jnp.maximum(
        lax.dot_general(gf, _bd(W1_ref[:, :D]), dn, preferred_element_type=f32)
        + rc, 0.0)
    b2d = jnp.concatenate([b2_ref[...], b2_ref[...]], axis=1)
    oh_ = jnp.maximum(
        lax.dot_general(x1, _bd(W2_ref[...]), dn, preferred_element_type=f32)
        + b2d, 0.0)

    p = lax.dot_general(oh_, _bd(A1_ref[:, :D]), dn, preferred_element_type=f32)
    a1bd = jnp.concatenate([a1b_ref[...], a1b_ref[...]], axis=1)
    uc = lax.dot_general(u_ref[...], _bd(A1_ref[:, D:]), dn,
                         preferred_element_type=f32) + a1bd
    a1 = jnp.maximum(p.reshape(L, bp, 2 * D) + uc[None], 0.0)
    a2bd = jnp.concatenate([a2b_ref[...], a2b_ref[...]], axis=1)
    a2 = jnp.maximum(
        lax.dot_general(a1.reshape(r, 2 * D), _bd(A2_ref[...]), dn,
                        preferred_element_type=f32) + a2bd, 0.0)

    A3d = jnp.concatenate([A3_ref[...], A3_ref[...]], axis=1)
    t2 = lax.dot_general(a2 * A3d, Sm, dn, preferred_element_type=f32)
    t3 = t2.reshape(L, bp, 2)
    m = jnp.max(t3, axis=0, keepdims=True)
    e = jnp.exp(t3 - m)
    w3 = e / jnp.sum(e, axis=0, keepdims=True)
    wf = lax.dot_general(w3.reshape(r, 2), Sm, (((1,), (0,)), ((), ())),
                         preferred_element_type=f32)
    out_ref[...] = jnp.sum((oh_ * wf).reshape(L, bp, 2 * D), axis=0)


def _tc_compute(g2, hrp, urep2, r2e_w, W1, b1, W2, b2, A1, a1b, A2, a2b, A3):
    bp = 128
    pairs = g2.shape[1]
    grid = pairs // bp
    full = lambda shape: pl.BlockSpec(shape, lambda i: tuple(0 for _ in shape))
    return pl.pallas_call(
        _tc_body,
        grid=(grid,),
        in_specs=[
            pl.BlockSpec((L, bp, 2 * D), lambda i: (0, i, 0)),
            pl.BlockSpec((bp, 2, L), lambda i: (i, 0, 0)),
            pl.BlockSpec((bp, 2 * D), lambda i: (i, 0)),
            full((5, D)),
            full((D, 2 * D)),
            full((1, D)),
            full((D, D)),
            full((1, D)),
            full((D, 2 * D)),
            full((1, D)),
            full((D, D)),
            full((1, D)),
            full((1, D)),
        ],
        out_specs=pl.BlockSpec((bp, 2 * D), lambda i: (i, 0)),
        out_shape=jax.ShapeDtypeStruct((pairs, 2 * D), jnp.float32),
    )(g2, hrp, urep2, r2e_w, W1, b1, W2, b2, A1, a1b, A2, a2b, A3)


def kernel(nodes, history_uv, history_r, history_uvt, v2e_w, u2e_w, r2e_w,
           t2e_w, W1, b1, W2, b2, A1, a1b, A2, a2b, A3, a3b):
    del history_uvt, t2e_w, a3b
    info = plsc.get_sparse_core_info()
    nw = info.num_cores * info.num_subcores
    nc = 2
    bsz = B // nc
    ch = bsz // nw
    hu_i = history_uv.astype(jnp.int32)
    hr_i = history_r.astype(jnp.int32)
    nodes_i = nodes.astype(jnp.int32)
    outs = []
    for c in range(nc):
        sl = slice(c * bsz, (c + 1) * bsz)
        hu_pages = jnp.transpose(hu_i[sl]) \
            .reshape(L, nw, ch).transpose(1, 0, 2)
        e_uv_flat, urep = _sc_gather(v2e_w, hu_pages, nodes_i[sl], u2e_w)
        g2 = e_uv_flat.reshape(L, bsz // 2, 2 * D)
        hrp = hr_i[sl].reshape(bsz // 2, 2, L)
        urep2 = urep.reshape(bsz // 2, 2 * D)
        outs.append(_tc_compute(
            g2, hrp, urep2, r2e_w, W1,
            b1.reshape(1, D), W2, b2.reshape(1, D),
            A1, a1b.reshape(1, D), A2, a2b.reshape(1, D), A3))
    return jnp.concatenate(outs, axis=0).reshape(B, D)


# speedup vs baseline: 2.4038x; 1.0014x over previous
import functools

import jax
import jax.numpy as jnp
from jax import lax
from jax.experimental import pallas as pl
from jax.experimental.pallas import tpu as pltpu
from jax.experimental.pallas import tpu_sc as plsc

B = 4096
L = 50
D = 64


def _sc_gather(table, hu_pages, nodes, utable):
    info = plsc.get_sparse_core_info()
    nw = info.num_cores * info.num_subcores
    ch = hu_pages.shape[2]
    bsz = nw * ch
    npw = bsz // nw

    @functools.partial(
        pl.kernel,
        mesh=plsc.VectorSubcoreMesh(core_axis_name="c", subcore_axis_name="s"),
        compiler_params=pltpu.CompilerParams(use_tc_tiling_on_sc=False),
        out_type=[
            jax.ShapeDtypeStruct((L * bsz, D), jnp.float32),
            jax.ShapeDtypeStruct((bsz, D), jnp.float32),
        ],
        scratch_types=[
            pltpu.VMEM((L, ch), jnp.int32),
            pltpu.VMEM((ch, D), jnp.float32),
            pltpu.VMEM((ch, D), jnp.float32),
            pltpu.VMEM((npw,), jnp.int32),
            pltpu.VMEM((npw, D), jnp.float32),
            pltpu.SemaphoreType.DMA,
            pltpu.SemaphoreType.DMA,
        ],
    )
    def k(table_hbm, hu_hbm, nodes_hbm, utable_hbm, out_hbm, uout_hbm,
          idxt_v, rb0, rb1, nidx_v, urows_v, sem0, sem1):
        wid = lax.axis_index("s") * info.num_cores + lax.axis_index("c")
        b0 = wid * ch

        pltpu.sync_copy(hu_hbm.at[wid], idxt_v)

        pltpu.async_copy(table_hbm.at[idxt_v.at[0]], rb0, sem0)

        def gbody(j, carry):
            l0 = 2 * j
            pltpu.async_copy(table_hbm.at[idxt_v.at[l0 + 1]], rb1, sem1)
            pltpu.make_async_copy(table_hbm.at[idxt_v.at[l0]], rb0, sem0).wait()
            pltpu.sync_copy(rb0, out_hbm.at[pl.ds(l0 * bsz + b0, ch)])

            @pl.when(j < (L // 2 - 1))
            def _():
                pltpu.async_copy(table_hbm.at[idxt_v.at[l0 + 2]], rb0, sem0)

            pltpu.make_async_copy(table_hbm.at[idxt_v.at[l0 + 1]], rb1,
                                  sem1).wait()
            pltpu.sync_copy(rb1, out_hbm.at[pl.ds((l0 + 1) * bsz + b0, ch)])
            return carry

        lax.fori_loop(0, L // 2, gbody, 0)

        pltpu.sync_copy(nodes_hbm.at[pl.ds(wid * npw, npw)], nidx_v)
        pltpu.async_copy(utable_hbm.at[nidx_v], urows_v, sem0).wait()
        pltpu.sync_copy(urows_v, uout_hbm.at[pl.ds(wid * npw, npw)])

    return k(table, hu_pages, nodes, utable)


def _bd(x):
    z = jnp.zeros((D, D), jnp.float32)
    return jnp.concatenate(
        [jnp.concatenate([x, z], axis=1), jnp.concatenate([z, x], axis=1)],
        axis=0)


def _tc_body(g_ref, hrp_ref, u_ref, r2e_ref, W1_ref, b1_ref, W2_ref, b2_ref,
             A1_ref, a1b_ref, A2_ref, a2b_ref, A3_ref, out_ref):
    bp = out_ref.shape[0]
    r = L * bp
    dn = (((1,), (1,)), ((), ()))
    f32 = jnp.float32

    lane = lax.broadcasted_iota(jnp.int32, (2, 2 * D), 1)
    half = lax.broadcasted_iota(jnp.int32, (2, 2 * D), 0)
    Sm = jnp.where((lane < D) == (half == 0), 1.0, 0.0).astype(f32)

    r_tab = lax.dot_general(r2e_ref[...], W1_ref[:, D:], dn,
                            preferred_element_type=f32) + b1_ref[...]
    z5 = jnp.zeros((5, D), f32)
    Rt = jnp.concatenate(
        [jnp.concatenate([r_tab, z5], axis=1),
         jnp.concatenate([z5, r_tab], axis=1)], axis=0)
    col = lax.broadcasted_iota(jnp.int32, (2, 10), 1)
    hh = lax.broadcasted_iota(jnp.int32, (2, 10), 0)
    E = jnp.where((col < 5) == (hh == 0), 1.0, 0.0).astype(f32)
    rvals = jnp.where(col[:1] < 5, col[:1], col[:1] - 5).astype(f32)
    hf = jnp.transpose(hrp_ref[...].astype(f32), (2, 0, 1)).reshape(r, 2)
    hrep = lax.dot_general(hf, E, (((1,), (0,)), ((), ())),
                           preferred_element_type=f32)
    oh = jnp.where(hrep == rvals, 1.0, 0.0).astype(f32)
    rc = lax.dot_general(oh, Rt, (((1,), (0,)), ((), ())),
                         preferred_element_type=f32)

    gf = g_ref[...].reshape(r, 2 * D)
    x1 = jnp.maximum(
        lax.dot_general(gf, _bd(W1_ref[:, :D]), dn, preferred_element_type=f32)
        + rc, 0.0)
    b2d = jnp.concatenate([b2_ref[...], b2_ref[...]], axis=1)
    oh_ = jnp.maximum(
        lax.dot_general(x1, _bd(W2_ref[...]), dn, preferred_element_type=f32)
        + b2d, 0.0)

    p = lax.dot_general(oh_, _bd(A1_ref[:, :D]), dn, preferred_element_type=f32)
    a1bd = jnp.concatenate([a1b_ref[...], a1b_ref[...]], axis=1)
    uc = lax.dot_general(u_ref[...], _bd(A1_ref[:, D:]), dn,
                         preferred_element_type=f32) + a1bd
    a1 = jnp.maximum(p.reshape(L, bp, 2 * D) + uc[None], 0.0)
    a2bd = jnp.concatenate([a2b_ref[...], a2b_ref[...]], axis=1)
    a2 = jnp.maximum(
        lax.dot_general(a1.reshape(r, 2 * D), _bd(A2_ref[...]), dn,
                        preferred_element_type=f32) + a2bd, 0.0)

    A3d = jnp.concatenate([A3_ref[...], A3_ref[...]], axis=1)
    t2 = lax.dot_general(a2 * A3d, Sm, dn, preferred_element_type=f32)
    t3 = t2.reshape(L, bp, 2)
    m = jnp.max(t3, axis=0, keepdims=True)
    e = jnp.exp(t3 - m)
    w3 = e / jnp.sum(e, axis=0, keepdims=True)
    wf = lax.dot_general(w3.reshape(r, 2), Sm, (((1,), (0,)), ((), ())),
                         preferred_element_type=f32)
    out_ref[...] = jnp.sum((oh_ * wf).reshape(L, bp, 2 * D), axis=0)


def _tc_compute(g2, hrp, urep2, r2e_w, W1, b1, W2, b2, A1, a1b, A2, a2b, A3):
    bp = 128
    pairs = g2.shape[1]
    grid = pairs // bp
    full = lambda shape: pl.BlockSpec(shape, lambda i: tuple(0 for _ in shape))
    return pl.pallas_call(
        _tc_body,
        grid=(grid,),
        in_specs=[
            pl.BlockSpec((L, bp, 2 * D), lambda i: (0, i, 0)),
            pl.BlockSpec((bp, 2, L), lambda i: (i, 0, 0)),
            pl.BlockSpec((bp, 2 * D), lambda i: (i, 0)),
            full((5, D)),
            full((D, 2 * D)),
            full((1, D)),
            full((D, D)),
            full((1, D)),
            full((D, 2 * D)),
            full((1, D)),
            full((D, D)),
            full((1, D)),
            full((1, D)),
        ],
        out_specs=pl.BlockSpec((bp, 2 * D), lambda i: (i, 0)),
        out_shape=jax.ShapeDtypeStruct((pairs, 2 * D), jnp.float32),
    )(g2, hrp, urep2, r2e_w, W1, b1, W2, b2, A1, a1b, A2, a2b, A3)


def kernel(nodes, history_uv, history_r, history_uvt, v2e_w, u2e_w, r2e_w,
           t2e_w, W1, b1, W2, b2, A1, a1b, A2, a2b, A3, a3b):
    del history_uvt, t2e_w, a3b
    info = plsc.get_sparse_core_info()
    nw = info.num_cores * info.num_subcores
    nc = 4
    bsz = B // nc
    ch = bsz // nw
    hu_i = history_uv.astype(jnp.int32)
    hr_i = history_r.astype(jnp.int32)
    nodes_i = nodes.astype(jnp.int32)
    outs = []
    for c in range(nc):
        sl = slice(c * bsz, (c + 1) * bsz)
        hu_pages = jnp.transpose(hu_i[sl]) \
            .reshape(L, nw, ch).transpose(1, 0, 2)
        e_uv_flat, urep = _sc_gather(v2e_w, hu_pages, nodes_i[sl], u2e_w)
        g2 = e_uv_flat.reshape(L, bsz // 2, 2 * D)
        hrp = hr_i[sl].reshape(bsz // 2, 2, L)
        urep2 = urep.reshape(bsz // 2, 2 * D)
        outs.append(_tc_compute(
            g2, hrp, urep2, r2e_w, W1,
            b1.reshape(1, D), W2, b2.reshape(1, D),
            A1, a1b.reshape(1, D), A2, a2b.reshape(1, D), A3))
    return jnp.concatenate(outs, axis=0).reshape(B, D)
